# baseline (speedup 1.0000x reference)
import functools

import jax
import jax.numpy as jnp
from jax import lax
from jax.experimental import pallas as pl
from jax.experimental.pallas import tpu as pltpu

S = 4096
LAT = 1024
SPLIT = 3072
R = 512
NBLK = S // R
SPLIT_BLK = SPLIT // R


def _tc_body(scal_ref, hist_ref, hval_ref, out_ref, carry, acc):
    i = pl.program_id(0)
    idx = scal_ref[0]
    ovf = scal_ref[1 + 2 * idx] == SPLIT
    del idx

    blk = hist_ref[0]
    shifted = jnp.logical_or(i < SPLIT_BLK, ovf)

    @pl.when(shifted)
    def _():
        first = jnp.where(i == 0, hval_ref[...], carry[...])
        is_mean_row = jnp.logical_and(i == SPLIT_BLK, ovf)
        first = jnp.where(is_mean_row, acc[...] * (1.0 / SPLIT), first)
        out_ref[0:1, :] = first
        out_ref[1:R, :] = blk[0 : R - 1, :]
        carry[...] = blk[R - 1 : R, :]
        @pl.when(jnp.logical_and(ovf, i < SPLIT_BLK))
        def _():
            part = first + jnp.sum(blk[0 : R - 1, :], axis=0, keepdims=True)
            acc[...] = jnp.where(i == 0, part, acc[...] + part)

    @pl.when(jnp.logical_not(shifted))
    def _():
        out_ref[...] = blk


def _tc_call(hist3, hval, scal):
    grid_spec = pltpu.PrefetchScalarGridSpec(
        num_scalar_prefetch=1,
        grid=(NBLK,),
        in_specs=[
            pl.BlockSpec((1, R, LAT), lambda i, s: (s[0], i, 0)),
            pl.BlockSpec((1, LAT), lambda i, s: (0, 0)),
        ],
        out_specs=pl.BlockSpec((R, LAT), lambda i, s: (i, 0)),
        scratch_shapes=[
            pltpu.VMEM((1, LAT), jnp.float32),
            pltpu.VMEM((1, LAT), jnp.float32),
        ],
    )
    return pl.pallas_call(
        _tc_body,
        grid_spec=grid_spec,
        out_shape=jax.ShapeDtypeStruct((S, LAT), jnp.float32),
    )(scal, hist3, hval)


def kernel(hist, hist_time, hist_val, hist_time_val, counter, index):
    hist3 = hist.reshape(hist.shape[0], S, LAT)
    scal = jnp.concatenate(
        [jnp.asarray(index, jnp.int32).reshape(1), counter.reshape(-1)]
    )
    out = _tc_call(hist3, hist_val, scal)
    return out.reshape(S, 1, LAT)


# speedup vs baseline: 8.4621x; 8.4621x over previous
import jax
import jax.numpy as jnp
from jax.experimental import pallas as pl
from jax.experimental.pallas import tpu as pltpu

S = 4096
LAT = 1024
SPLIT = 3072
W = 128
G = LAT // W
SR = S * G
SPLITR = SPLIT * G
R = 4096
NBLK = SR // R
SPLIT_BLK = SPLITR // R


def _tc_body(scal_ref, hist_ref, hval_ref, out_ref, carry, acc):
    i = pl.program_id(0)
    idx = scal_ref[0]
    ovf = scal_ref[1 + 2 * idx] == SPLIT

    blk = hist_ref[0]
    shifted = jnp.logical_or(i < SPLIT_BLK, ovf)

    @pl.when(shifted)
    def _():
        first = jnp.where(i == 0, hval_ref[...], carry[...])
        is_mean_row = jnp.logical_and(i == SPLIT_BLK, ovf)
        first = jnp.where(is_mean_row, acc[...] * (1.0 / SPLIT), first)
        out_ref[0:G, :] = first
        out_ref[G:R, :] = blk[0 : R - G, :]
        carry[...] = blk[R - G : R, :]
        @pl.when(jnp.logical_and(ovf, i < SPLIT_BLK))
        def _():
            part = first + jnp.sum(
                blk[0 : R - G, :].reshape(R // G - 1, G, W), axis=0
            )
            acc[...] = jnp.where(i == 0, part, acc[...] + part)

    @pl.when(jnp.logical_not(shifted))
    def _():
        out_ref[...] = blk


def _tc_call(histr, hvalr, scal):
    grid_spec = pltpu.PrefetchScalarGridSpec(
        num_scalar_prefetch=1,
        grid=(NBLK,),
        in_specs=[
            pl.BlockSpec((1, R, W), lambda i, s: (s[0], i, 0)),
            pl.BlockSpec((G, W), lambda i, s: (0, 0)),
        ],
        out_specs=pl.BlockSpec((R, W), lambda i, s: (i, 0)),
        scratch_shapes=[
            pltpu.VMEM((G, W), jnp.float32),
            pltpu.VMEM((G, W), jnp.float32),
        ],
    )
    return pl.pallas_call(
        _tc_body,
        grid_spec=grid_spec,
        out_shape=jax.ShapeDtypeStruct((SR, W), jnp.float32),
    )(scal, histr, hvalr)


def kernel(hist, hist_time, hist_val, hist_time_val, counter, index):
    histr = hist.reshape(hist.shape[0], SR, W)
    hvalr = hist_val.reshape(G, W)
    scal = jnp.concatenate(
        [jnp.asarray(index, jnp.int32).reshape(1), counter.reshape(-1)]
    )
    out = _tc_call(histr, hvalr, scal)
    return out.reshape(S, 1, LAT)
